# baseline (speedup 1.0000x reference)
import functools

import jax
import jax.numpy as jnp
from jax import lax
from jax.experimental import pallas as pl
from jax.experimental.pallas import tpu as pltpu
from jax.experimental.pallas import tpu_sc as plsc

N_A = 259200
N_B = 65160
NNZ = 4
BC = 128
DST_SHAPE = (181, 360)

CH = 64
F_CORE0 = 32
N_PAD = 65536
NCHUNKS = N_PAD // CH


def _regrid_sc(xT, col_r, w_r):
    info = plsc.get_sparse_core_info()
    nc, ns = info.num_cores, info.num_subcores
    pair = NCHUNKS // ns
    f0 = F_CORE0
    maxw = max(f0, pair - f0)
    wstride = NNZ * CH
    mesh = plsc.VectorSubcoreMesh(core_axis_name="c", subcore_axis_name="s")

    @functools.partial(
        pl.kernel,
        mesh=mesh,
        compiler_params=pltpu.CompilerParams(needs_layout_passes=False),
        out_type=jax.ShapeDtypeStruct((BC, N_PAD), jnp.float32),
        scratch_types=[
            pltpu.VMEM((maxw, NNZ, CH), jnp.int32),
            pltpu.VMEM((maxw * NNZ * CH,), jnp.float32),
            pltpu.VMEM((2, NNZ, CH, BC), jnp.float32),
            pltpu.VMEM((2 * BC, 2 * CH), jnp.float32),
            pltpu.SemaphoreType.DMA,
            pltpu.SemaphoreType.DMA,
            pltpu.SemaphoreType.DMA,
            pltpu.SemaphoreType.DMA,
        ],
    )
    def k(xT_h, col_h, w_h, out_h, idx_v, w_v, rows_v, acc_v,
          g0, g1, s0, s1):
        gsem = (g0, g1)
        ssem = (s0, s1)
        cax = lax.axis_index("c")
        c0 = lax.axis_index("s") * pair + jnp.where(cax == 0, 0, f0)
        n_my = jnp.where(cax == 0, f0, pair - f0)
        iota = lax.iota(jnp.int32, 16)

        pltpu.sync_copy(col_h.at[pl.ds(c0, maxw)], idx_v)
        pltpu.sync_copy(w_h.at[pl.ds(c0 * wstride, maxw * wstride)], w_v)

        def fire(ci, b):
            for kk in range(NNZ):
                pltpu.async_copy(
                    xT_h.at[idx_v.at[ci, kk]], rows_v.at[b, kk], gsem[b])

        def gwait(b):
            for kk in range(NNZ):
                pltpu.make_async_copy(
                    xT_h.at[idx_v.at[0, kk]], rows_v.at[b, kk], gsem[b]).wait()

        fire(0, 0)
        fire(1, 1)

        def quad_body(i2, carry):
            for q in range(4):
                c = 4 * i2 + q
                bg = q % 2
                p = q // 2

                if q in (0, 2):
                    @pl.when(i2 >= 1)
                    def _wait_store(_p=p):
                        pltpu.make_async_copy(
                            acc_v.at[pl.ds(_p * BC, BC)],
                            out_h.at[:, pl.ds(0, 2 * CH)], ssem[_p]).wait()

                gwait(bg)
                wbase = c * wstride

                @plsc.parallel_loop(0, CH, 1, unroll=4)
                def dst_body(j, _bg=bg, _p=p, _h=q % 2, _wbase=wbase):
                    wsp = [
                        plsc.load_gather(
                            w_v,
                            [jnp.full((16,), kk * CH, jnp.int32) + (_wbase + j)])
                        for kk in range(NNZ)
                    ]
                    cidx = jnp.full((16,), _h * CH, jnp.int32) + j
                    for f in range(BC // 16):
                        sl = pl.ds(f * 16, 16)
                        r01 = (wsp[0] * rows_v[_bg, 0, j, sl]
                               + wsp[1] * rows_v[_bg, 1, j, sl])
                        r23 = (wsp[2] * rows_v[_bg, 2, j, sl]
                               + wsp[3] * rows_v[_bg, 3, j, sl])
                        ridx = iota + (_p * BC + f * 16)
                        plsc.store_scatter(acc_v, [ridx, cidx], r01 + r23)

                @pl.when(c + 2 < n_my)
                def _fire_next(_c=c, _b=bg):
                    fire(_c + 2, _b)

                if q in (1, 3):
                    pltpu.async_copy(
                        acc_v.at[pl.ds(p * BC, BC)],
                        out_h.at[:, pl.ds((c0 + 4 * i2 + 2 * p) * CH, 2 * CH)],
                        ssem[p])
            return carry

        lax.fori_loop(0, n_my // 4, quad_body, 0)

        for b in range(2):
            pltpu.make_async_copy(
                acc_v.at[pl.ds(b * BC, BC)],
                out_h.at[:, pl.ds(0, 2 * CH)], ssem[b]).wait()

    return k(xT, col_r, w_r)


def kernel(x, row, col, weights):
    lead_shape = x.shape[:-2]
    xT = jnp.transpose(x.reshape(-1, *x.shape[-2:]), (1, 2, 0)).reshape(N_A, -1)

    pad = N_PAD - N_B
    colp = jnp.concatenate([col, jnp.zeros((pad * NNZ,), jnp.int32)])
    wp = jnp.concatenate([weights, jnp.zeros((pad * NNZ,), jnp.float32)])
    col_r = colp.reshape(NCHUNKS, CH, NNZ).transpose(0, 2, 1)
    w_r = wp.reshape(NCHUNKS, CH, NNZ).transpose(0, 2, 1).reshape(-1)

    y = _regrid_sc(xT, col_r, w_r)
    ny, nx = DST_SHAPE
    return y[:, :N_B].reshape(*lead_shape, ny, nx)


# speedup vs baseline: 1.0576x; 1.0576x over previous
import functools

import jax
import jax.numpy as jnp
from jax import lax
from jax.experimental import pallas as pl
from jax.experimental.pallas import tpu as pltpu
from jax.experimental.pallas import tpu_sc as plsc

N_A = 259200
N_B = 65160
NNZ = 4
BC = 128
DST_SHAPE = (181, 360)

CH = 64
N_PAD = 65536
NCHUNKS = N_PAD // CH


def _regrid_sc(xT, col_r, w_r):
    info = plsc.get_sparse_core_info()
    nc, ns = info.num_cores, info.num_subcores
    nw = nc * ns
    per_w = NCHUNKS // nw
    wstride = NNZ * CH
    mesh = plsc.VectorSubcoreMesh(core_axis_name="c", subcore_axis_name="s")

    @functools.partial(
        pl.kernel,
        mesh=mesh,
        compiler_params=pltpu.CompilerParams(needs_layout_passes=False),
        out_type=jax.ShapeDtypeStruct((N_PAD, BC), jnp.float32),
        scratch_types=[
            pltpu.VMEM((per_w, NNZ, CH), jnp.int32),
            pltpu.VMEM((per_w * NNZ * CH,), jnp.float32),
            pltpu.VMEM((2, NNZ, CH, BC), jnp.float32),
            pltpu.VMEM((2, CH, BC), jnp.float32),
            pltpu.SemaphoreType.DMA,
            pltpu.SemaphoreType.DMA,
            pltpu.SemaphoreType.DMA,
            pltpu.SemaphoreType.DMA,
        ],
    )
    def k(xT_h, col_h, w_h, out_h, idx_v, w_v, rows_v, acc_v,
          g0, g1, s0, s1):
        gsem = (g0, g1)
        ssem = (s0, s1)
        wid = lax.axis_index("s") * nc + lax.axis_index("c")
        c0 = wid * per_w

        pltpu.sync_copy(col_h.at[pl.ds(c0, per_w)], idx_v)
        pltpu.sync_copy(w_h.at[pl.ds(c0 * wstride, per_w * wstride)], w_v)

        def fire(ci, b):
            for kk in range(NNZ):
                pltpu.async_copy(
                    xT_h.at[idx_v.at[ci, kk]], rows_v.at[b, kk], gsem[b])

        def gwait(b):
            for kk in range(NNZ):
                pltpu.make_async_copy(
                    xT_h.at[idx_v.at[0, kk]], rows_v.at[b, kk], gsem[b]).wait()

        fire(0, 0)
        fire(1, 1)

        def pair_body(i, carry):
            for half in range(2):
                c = 2 * i + half

                @pl.when(c >= 2)
                def _wait_store(_b=half):
                    pltpu.make_async_copy(
                        acc_v.at[_b], out_h.at[pl.ds(0, CH)], ssem[_b]).wait()

                gwait(half)
                wbase = c * wstride

                @plsc.parallel_loop(0, CH, 1, unroll=8)
                def dst_body(j, _b=half, _wbase=wbase):
                    wsp = [
                        plsc.load_gather(
                            w_v,
                            [jnp.full((16,), kk * CH, jnp.int32) + (_wbase + j)])
                        for kk in range(NNZ)
                    ]
                    for f in range(BC // 16):
                        sl = pl.ds(f * 16, 16)
                        r01 = (wsp[0] * rows_v[_b, 0, j, sl]
                               + wsp[1] * rows_v[_b, 1, j, sl])
                        r23 = (wsp[2] * rows_v[_b, 2, j, sl]
                               + wsp[3] * rows_v[_b, 3, j, sl])
                        acc_v[_b, j, sl] = r01 + r23

                @pl.when(c + 2 < per_w)
                def _fire_next(_c=c, _b=half):
                    fire(_c + 2, _b)

                pltpu.async_copy(
                    acc_v.at[half], out_h.at[pl.ds((c0 + c) * CH, CH)],
                    ssem[half])
            return carry

        lax.fori_loop(0, per_w // 2, pair_body, 0)

        for b in range(2):
            pltpu.make_async_copy(
                acc_v.at[b], out_h.at[pl.ds(0, CH)], ssem[b]).wait()

    return k(xT, col_r, w_r)


def kernel(x, row, col, weights):
    lead_shape = x.shape[:-2]
    xT = jnp.transpose(x.reshape(-1, *x.shape[-2:]), (1, 2, 0)).reshape(N_A, -1)

    pad = N_PAD - N_B
    colp = jnp.concatenate([col, jnp.zeros((pad * NNZ,), jnp.int32)])
    wp = jnp.concatenate([weights, jnp.zeros((pad * NNZ,), jnp.float32)])
    col_r = colp.reshape(NCHUNKS, CH, NNZ).transpose(0, 2, 1)
    w_r = wp.reshape(NCHUNKS, CH, NNZ).transpose(0, 2, 1).reshape(-1)

    yT = _regrid_sc(xT, col_r, w_r)
    y = yT[:N_B].T
    ny, nx = DST_SHAPE
    return y.reshape(*lead_shape, ny, nx)


# speedup vs baseline: 1.1183x; 1.0574x over previous
import functools

import jax
import jax.numpy as jnp
from jax import lax
from jax.experimental import pallas as pl
from jax.experimental.pallas import tpu as pltpu
from jax.experimental.pallas import tpu_sc as plsc

N_A = 259200
N_B = 65160
NNZ = 4
BC = 128
DST_SHAPE = (181, 360)

CH = 64
N_PAD = 65536
NCHUNKS = N_PAD // CH


def _regrid_sc(xT, col_r, w_r):
    info = plsc.get_sparse_core_info()
    nc, ns = info.num_cores, info.num_subcores
    nw = nc * ns
    per_w = NCHUNKS // nw
    wstride = NNZ * CH
    mesh = plsc.VectorSubcoreMesh(core_axis_name="c", subcore_axis_name="s")

    @functools.partial(
        pl.kernel,
        mesh=mesh,
        compiler_params=pltpu.CompilerParams(needs_layout_passes=False),
        out_type=jax.ShapeDtypeStruct((N_PAD, BC), jnp.float32),
        scratch_types=[
            pltpu.VMEM((per_w, NNZ, CH), jnp.int32),
            pltpu.VMEM((per_w * NNZ * CH,), jnp.float32),
            pltpu.VMEM((2, NNZ, CH, BC), jnp.float32),
            pltpu.VMEM((2, CH, BC), jnp.float32),
            pltpu.SemaphoreType.DMA,
            pltpu.SemaphoreType.DMA,
            pltpu.SemaphoreType.DMA,
            pltpu.SemaphoreType.DMA,
        ],
    )
    def k(xT_h, col_h, w_h, out_h, idx_v, w_v, rows_v, acc_v,
          g0, g1, s0, s1):
        gsem = (g0, g1)
        ssem = (s0, s1)
        wid = lax.axis_index("s") * nc + lax.axis_index("c")
        c0 = wid * per_w

        pltpu.sync_copy(col_h.at[pl.ds(c0, per_w)], idx_v)
        pltpu.sync_copy(w_h.at[pl.ds(c0 * wstride, per_w * wstride)], w_v)

        def fire(ci, b):
            for kk in range(NNZ):
                pltpu.async_copy(
                    xT_h.at[idx_v.at[ci, kk]], rows_v.at[b, kk], gsem[b])

        def gwait(b):
            for kk in range(NNZ):
                pltpu.make_async_copy(
                    xT_h.at[idx_v.at[0, kk]], rows_v.at[b, kk], gsem[b]).wait()

        fire(0, 0)
        fire(1, 1)

        def pair_body(i, carry):
            for half in range(2):
                c = 2 * i + half

                @pl.when(c >= 2)
                def _wait_store(_b=half):
                    pltpu.make_async_copy(
                        acc_v.at[_b], out_h.at[pl.ds(0, CH)], ssem[_b]).wait()

                gwait(half)
                wbase = c * wstride

                @plsc.parallel_loop(0, CH, 1, unroll=4)
                def dst_body(j, _b=half, _wbase=wbase):
                    wsp = [
                        plsc.load_gather(
                            w_v,
                            [jnp.full((16,), kk * CH, jnp.int32) + (_wbase + j)])
                        for kk in range(NNZ)
                    ]
                    for f in range(BC // 16):
                        sl = pl.ds(f * 16, 16)
                        r01 = (wsp[0] * rows_v[_b, 0, j, sl]
                               + wsp[1] * rows_v[_b, 1, j, sl])
                        r23 = (wsp[2] * rows_v[_b, 2, j, sl]
                               + wsp[3] * rows_v[_b, 3, j, sl])
                        acc_v[_b, j, sl] = r01 + r23

                @pl.when(c + 2 < per_w)
                def _fire_next(_c=c, _b=half):
                    fire(_c + 2, _b)

                pltpu.async_copy(
                    acc_v.at[half], out_h.at[pl.ds((c0 + c) * CH, CH)],
                    ssem[half])
            return carry

        lax.fori_loop(0, per_w // 2, pair_body, 0)

        for b in range(2):
            pltpu.make_async_copy(
                acc_v.at[b], out_h.at[pl.ds(0, CH)], ssem[b]).wait()

    return k(xT, col_r, w_r)


def kernel(x, row, col, weights):
    lead_shape = x.shape[:-2]
    xT = jnp.transpose(x.reshape(-1, *x.shape[-2:]), (1, 2, 0)).reshape(N_A, -1)

    pad = N_PAD - N_B
    colp = jnp.concatenate([col, jnp.zeros((pad * NNZ,), jnp.int32)])
    wp = jnp.concatenate([weights, jnp.zeros((pad * NNZ,), jnp.float32)])
    col_r = colp.reshape(NCHUNKS, CH, NNZ).transpose(0, 2, 1)
    w_r = wp.reshape(NCHUNKS, CH, NNZ).transpose(0, 2, 1).reshape(-1)

    yT = _regrid_sc(xT, col_r, w_r)
    y = yT[:N_B].T
    ny, nx = DST_SHAPE
    return y.reshape(*lead_shape, ny, nx)


# speedup vs baseline: 1.1184x; 1.0001x over previous
import functools

import jax
import jax.numpy as jnp
from jax import lax
from jax.experimental import pallas as pl
from jax.experimental.pallas import tpu as pltpu
from jax.experimental.pallas import tpu_sc as plsc

N_A = 259200
N_B = 65160
NNZ = 4
BC = 128
DST_SHAPE = (181, 360)

CH = 64
N_PAD = 65536
NCHUNKS = N_PAD // CH


def _regrid_sc(xT, col_r, w_r):
    info = plsc.get_sparse_core_info()
    nc, ns = info.num_cores, info.num_subcores
    nw = nc * ns
    per_w = NCHUNKS // nw
    wstride = NNZ * CH
    mesh = plsc.VectorSubcoreMesh(core_axis_name="c", subcore_axis_name="s")

    @functools.partial(
        pl.kernel,
        mesh=mesh,
        compiler_params=pltpu.CompilerParams(needs_layout_passes=False),
        out_type=jax.ShapeDtypeStruct((N_PAD, BC), jnp.float32),
        scratch_types=[
            pltpu.VMEM((per_w, NNZ, CH), jnp.int32),
            pltpu.VMEM((per_w * NNZ * CH,), jnp.float32),
            pltpu.VMEM((2, NNZ, CH, BC), jnp.float32),
            pltpu.VMEM((2, CH, BC), jnp.float32),
            pltpu.SemaphoreType.DMA,
            pltpu.SemaphoreType.DMA,
            pltpu.SemaphoreType.DMA,
            pltpu.SemaphoreType.DMA,
        ],
    )
    def k(xT_h, col_h, w_h, out_h, idx_v, w_v, rows_v, acc_v,
          g0, g1, s0, s1):
        gsem = (g0, g1)
        ssem = (s0, s1)
        wid = lax.axis_index("s") * nc + lax.axis_index("c")
        c0 = wid * per_w

        pltpu.sync_copy(col_h.at[pl.ds(c0, per_w)], idx_v)
        pltpu.sync_copy(w_h.at[pl.ds(c0 * wstride, per_w * wstride)], w_v)

        def fire(ci, b):
            for kk in range(NNZ):
                pltpu.async_copy(
                    xT_h.at[idx_v.at[ci, kk]], rows_v.at[b, kk], gsem[b])

        def gwait(b):
            for kk in range(NNZ):
                pltpu.make_async_copy(
                    xT_h.at[idx_v.at[0, kk]], rows_v.at[b, kk], gsem[b]).wait()

        fire(0, 0)
        fire(1, 1)

        def pair_body(i, carry):
            for half in range(2):
                c = 2 * i + half

                @pl.when(c >= 2)
                def _wait_store(_b=half):
                    pltpu.make_async_copy(
                        acc_v.at[_b], out_h.at[pl.ds(0, CH)], ssem[_b]).wait()

                gwait(half)
                wbase = c * wstride

                @plsc.parallel_loop(0, CH, 1, unroll=2)
                def dst_body(j, _b=half, _wbase=wbase):
                    wsp = [
                        plsc.load_gather(
                            w_v,
                            [jnp.full((16,), kk * CH, jnp.int32) + (_wbase + j)])
                        for kk in range(NNZ)
                    ]
                    for f in range(BC // 16):
                        sl = pl.ds(f * 16, 16)
                        r01 = (wsp[0] * rows_v[_b, 0, j, sl]
                               + wsp[1] * rows_v[_b, 1, j, sl])
                        r23 = (wsp[2] * rows_v[_b, 2, j, sl]
                               + wsp[3] * rows_v[_b, 3, j, sl])
                        acc_v[_b, j, sl] = r01 + r23

                @pl.when(c + 2 < per_w)
                def _fire_next(_c=c, _b=half):
                    fire(_c + 2, _b)

                pltpu.async_copy(
                    acc_v.at[half], out_h.at[pl.ds((c0 + c) * CH, CH)],
                    ssem[half])
            return carry

        lax.fori_loop(0, per_w // 2, pair_body, 0)

        for b in range(2):
            pltpu.make_async_copy(
                acc_v.at[b], out_h.at[pl.ds(0, CH)], ssem[b]).wait()

    return k(xT, col_r, w_r)


def kernel(x, row, col, weights):
    lead_shape = x.shape[:-2]
    xT = jnp.transpose(x.reshape(-1, *x.shape[-2:]), (1, 2, 0)).reshape(N_A, -1)

    pad = N_PAD - N_B
    colp = jnp.concatenate([col, jnp.zeros((pad * NNZ,), jnp.int32)])
    wp = jnp.concatenate([weights, jnp.zeros((pad * NNZ,), jnp.float32)])
    col_r = colp.reshape(NCHUNKS, CH, NNZ).transpose(0, 2, 1)
    w_r = wp.reshape(NCHUNKS, CH, NNZ).transpose(0, 2, 1).reshape(-1)

    yT = _regrid_sc(xT, col_r, w_r)
    y = yT[:N_B].T
    ny, nx = DST_SHAPE
    return y.reshape(*lead_shape, ny, nx)
